# baseline (speedup 1.0000x reference)
import functools

import jax
import jax.numpy as jnp
from jax import lax
from jax.experimental import pallas as pl
from jax.experimental.pallas import tpu as pltpu
from jax.experimental.pallas import tpu_sc as plsc

B, N, F, D = 64, 256, 9, 768
NC, NS, L = 2, 16, 16
NW = NC * NS
NPW = N // NW
C = 64
HB = B // C
NCHUNK = NPW * HB
NSLOT = 2
OUT_ROWS = (N + 1) * B

_mesh = plsc.VectorSubcoreMesh(core_axis_name="c", subcore_axis_name="s")


@functools.partial(
    pl.kernel,
    out_type=jax.ShapeDtypeStruct((OUT_ROWS, D), jnp.float32),
    mesh=_mesh,
    compiler_params=pltpu.CompilerParams(use_tc_tiling_on_sc=False),
    scratch_types=[
        pltpu.VMEM((NPW, F, HB, C), jnp.int32),
        pltpu.VMEM((NPW, HB, C), jnp.int32),
        pltpu.VMEM((NPW, HB, C), jnp.int32),
        pltpu.VMEM((NSLOT, C, D), jnp.float32),
        pltpu.VMEM((2, D), jnp.float32),
        [pltpu.SemaphoreType.DMA] * NSLOT,
        [pltpu.SemaphoreType.DMA] * NSLOT,
    ],
)
def _sc_body(x_hbm, ind_hbm, outd_hbm, atab, itab, otab, tok, out_hbm,
             x_v, ind_v, outd_v, acc, tok_v, semg, semo):
    w = lax.axis_index("s") * NC + lax.axis_index("c")

    pltpu.sync_copy(x_hbm.at[w], x_v)
    pltpu.sync_copy(ind_hbm.at[w], ind_v)
    pltpu.sync_copy(outd_hbm.at[w], outd_v)
    pltpu.sync_copy(tok, tok_v.at[pl.ds(0, 1)])
    pltpu.sync_copy(tok, tok_v.at[pl.ds(1, 1)])
    pltpu.sync_copy(tok_v, out_hbm.at[pl.ds(2 * w, 2)])

    def zero_slot(p):
        z = jnp.zeros((L,), jnp.float32)

        @pl.loop(0, D // L)
        def _z(j):
            sl = pl.ds(j * L, L)
            for i in range(C):
                acc[p, i, sl] = z

    def gathers(c, p):
        k = c // HB
        h = c % HB
        copies = [pltpu.make_async_copy(atab.at[x_v.at[k, f, h]], acc.at[p],
                                        semg[p]) for f in range(F)]
        copies.append(pltpu.make_async_copy(itab.at[ind_v.at[k, h]], acc.at[p],
                                            semg[p]))
        copies.append(pltpu.make_async_copy(otab.at[outd_v.at[k, h]],
                                            acc.at[p], semg[p]))
        return copies

    def fire_gathers(c, p):
        k = c // HB
        h = c % HB
        for f in range(F):
            pltpu.async_copy(atab.at[x_v.at[k, f, h]], acc.at[p], semg[p],
                             add=True)
        pltpu.async_copy(itab.at[ind_v.at[k, h]], acc.at[p], semg[p],
                         add=True)
        pltpu.async_copy(otab.at[outd_v.at[k, h]], acc.at[p], semg[p],
                         add=True)

    def wait_gathers(c, p):
        for cp in gathers(c, p):
            cp.wait()

    def out_copy(c, p):
        row0 = (w * NPW + c // HB + 1) * B + (c % HB) * C
        return pltpu.make_async_copy(
            acc.at[p], out_hbm.at[pl.ds(row0, C)], semo[p])

    zero_slot(0)
    fire_gathers(0, 0)

    @pl.loop(0, NCHUNK, step=NSLOT)
    def _cs(c0):
        for p in range(NSLOT):
            c = c0 + p
            p1 = (p + 1) % NSLOT

            @pl.when(c + 1 < NCHUNK)
            def _():
                @pl.when(c + 1 >= NSLOT)
                def _():
                    out_copy(c + 1 - NSLOT, p1).wait()

                zero_slot(p1)
                fire_gathers(c + 1, p1)

            wait_gathers(c, p)
            out_copy(c, p).start()

    for k in range(NSLOT):
        out_copy(NCHUNK - NSLOT + k, k).wait()


def kernel(x, in_degree, out_degree, atom_table, in_deg_table, out_deg_table,
           graph_token):
    x5 = x.transpose(1, 2, 0).reshape(NW, NPW, F, HB, C)
    ind4 = in_degree.transpose(1, 0).reshape(NW, NPW, HB, C)
    outd4 = out_degree.transpose(1, 0).reshape(NW, NPW, HB, C)
    out = _sc_body(x5, ind4, outd4, atom_table, in_deg_table, out_deg_table,
                   graph_token)
    return out.reshape(N + 1, B, D).transpose(1, 0, 2)
